# Initial kernel scaffold; baseline (speedup 1.0000x reference)
import jax
import jax.numpy as jnp
from jax.experimental import pallas as pl


def kernel(feats, inp_points, out_points, out_extents, scale_compat, neighbors_index, neighbors_row_splits, neighbors_distance, W, b):
    raise NotImplementedError("write your pallas kernel here")


# speedup vs baseline: 3.1730x; 3.1730x over previous
import functools

import jax
import jax.numpy as jnp
from jax import lax
from jax.experimental import pallas as pl
from jax.experimental.pallas import tpu as pltpu
from jax.experimental.pallas import tpu_sc as plsc

_N_IN = 10000
_M = 5000
_E = 160000
_C = 128
_K = 4
_NCELL = _K * _K * _K
_AW = _NCELL * _C
_GW = 144
_NW = 32
_MPW = 160
_M_PAD = _NW * _MPW
_RS_PAD = (_NW - 1) * _MPW + _MPW + 8
_EW = 136
_E_PAD = _E + _EW
_CH = 128
_BM = 200


def _rsqrt(x):
    xi = plsc.bitcast(x, jnp.int32)
    y = plsc.bitcast(jnp.int32(0x5F3759DF) - (xi >> 1), jnp.float32)
    for _ in range(3):
        y = y * (1.5 - 0.5 * x * y * y)
    return y


def _sc_body(gtab, rs, eidx, esc, edist, opts, ext,
             a_out, norm_out,
             rs_v, op_v, ext_v, ew_idx, ew_sc, ew_dist, gidx, gbuf,
             acc, sstage, cstage, normbuf, sem):
    cid = lax.axis_index("c")
    sid = lax.axis_index("s")
    wid = sid * 2 + cid
    m0 = pl.multiple_of(wid * _MPW, _MPW)
    mcnt = jnp.minimum(_MPW, _M - m0)

    pltpu.sync_copy(rs.at[pl.ds(m0, _MPW + 8)], rs_v.at[pl.ds(0, _MPW + 8)])
    pltpu.sync_copy(opts.at[pl.ds(m0, _MPW)], op_v)
    pltpu.sync_copy(ext, ext_v)
    inv_h = 1.0 / (0.5 * ext_v[pl.ds(0, 16)])

    zero16 = jnp.zeros((16,), jnp.float32)
    iota16 = lax.iota(jnp.int32, 16)
    col_x = jnp.full((16,), _C, jnp.int32)
    col_y = jnp.full((16,), _C + 1, jnp.int32)
    col_z = jnp.full((16,), _C + 2, jnp.int32)

    def dst_body(dd, _carry):
        rsv = rs_v[pl.ds(dd, 16)]
        s = rsv[0]
        t = rsv[1]
        opv = op_v[dd, pl.ds(0, 16)]
        ox = opv[0]
        oy = opv[1]
        oz = opv[2]

        def zero_body(k, _):
            acc[pl.ds(k * 16, 16)] = zero16
            return 0
        lax.fori_loop(0, _AW // 16, zero_body, 0)

        nrounds = (t - s + (_CH - 1)) >> 7

        def round_body(rr, nacc):
            c0 = s + rr * _CH
            cl = jnp.minimum(t - c0, _CH)
            a0 = pl.multiple_of(c0 & ~jnp.int32(7), 8)
            off = c0 - a0
            pltpu.sync_copy(eidx.at[pl.ds(a0, _EW)], ew_idx)
            pltpu.sync_copy(esc.at[pl.ds(a0, _EW)], ew_sc)
            pltpu.sync_copy(edist.at[pl.ds(a0, _EW)], ew_dist)

            def cp_body(g, _):
                gidx[pl.ds(g * 16, 16)] = ew_idx[pl.ds(off + g * 16, 16)]
                return 0
            lax.fori_loop(0, _CH // 16, cp_body, 0)

            pltpu.async_copy(gtab.at[gidx], gbuf, sem).wait()

            ngroups = (cl + 15) >> 4

            def geom_body(g, na):
                lane0 = g * 16
                lanes = iota16 + lane0
                msk = lanes < cl
                x = plsc.load_gather(gbuf, [lanes, col_x])
                y = plsc.load_gather(gbuf, [lanes, col_y])
                z = plsc.load_gather(gbuf, [lanes, col_z])
                rx = (x - ox) * inv_h
                ry = (y - oy) * inv_h
                rz = (z - oz) * inv_h
                r2sq = rx * rx + ry * ry + rz * rz + 1e-12
                ih = _rsqrt(r2sq)
                r2 = r2sq * ih
                rinf = jnp.maximum(jnp.maximum(jnp.abs(rx), jnp.abs(ry)),
                                   jnp.abs(rz))
                scale = jnp.where(rinf > 1e-12,
                                  r2 / jnp.maximum(rinf, 1e-12), 0.0)
                cx = jnp.clip(rx * scale, -1.0, 1.0)
                cy = jnp.clip(ry * scale, -1.0, 1.0)
                cz = jnp.clip(rz * scale, -1.0, 1.0)
                gx = (cx * 0.5 + 0.5) * (_K - 1.0)
                gy = (cy * 0.5 + 0.5) * (_K - 1.0)
                gz = (cz * 0.5 + 0.5) * (_K - 1.0)
                g0x = jnp.minimum(gx.astype(jnp.int32), _K - 2)
                g0y = jnp.minimum(gy.astype(jnp.int32), _K - 2)
                g0z = jnp.minimum(gz.astype(jnp.int32), _K - 2)
                fx = gx - g0x.astype(jnp.float32)
                fy = gy - g0y.astype(jnp.float32)
                fz = gz - g0z.astype(jnp.float32)
                scv = ew_sc[pl.ds(off + lane0, 16)]
                dv = ew_dist[pl.ds(off + lane0, 16)]
                rsq = jnp.clip(dv, 0.0, 1.0)
                t1 = 1.0 - rsq
                imp = jnp.where(msk, scv * (t1 * t1 * t1), 0.0)
                na = na + jnp.sum(imp)
                wx = (1.0 - fx, fx)
                wy = (1.0 - fy, fy)
                wz = (1.0 - fz, fz)
                for c in range(8):
                    bx, by, bz = c & 1, (c >> 1) & 1, (c >> 2) & 1
                    sv = imp * (wx[bx] * wy[by] * wz[bz])
                    cell = ((g0x + bx) * (_K * _K) + (g0y + by) * _K
                            + (g0z + bz))
                    sstage[c, pl.ds(lane0, 16)] = sv
                    cstage[c, pl.ds(lane0, 16)] = cell
                return na
            nacc = lax.fori_loop(0, ngroups, geom_body, nacc)

            def edge_body(i, _):
                fvs = [gbuf[i, pl.ds(j * 16, 16)] for j in range(8)]
                for c in range(8):
                    sv = jnp.full((16,), sstage[c, pl.ds(i, 16)][0])
                    base = cstage[c, pl.ds(i, 16)][0] * _C
                    for j in range(8):
                        plsc.addupdate(acc.at[pl.ds(base + j * 16, 16)],
                                       fvs[j] * sv)
                return 0
            lax.fori_loop(0, cl, edge_body, 0)
            return nacc

        nsum = lax.fori_loop(0, nrounds, round_body, jnp.float32(0.0))
        normbuf[pl.ds(dd, 16)] = jnp.full((16,), nsum)
        pltpu.sync_copy(acc, a_out.at[m0 + dd])
        return 0

    lax.fori_loop(0, mcnt, dst_body, 0)
    pltpu.sync_copy(normbuf.at[pl.ds(0, _MPW)], norm_out.at[pl.ds(m0, _MPW)])


def _tc_body(a_ref, w_ref, n_ref, b_ref, o_ref):
    prod = jnp.dot(a_ref[...], w_ref[...],
                   preferred_element_type=jnp.float32)
    nv = n_ref[0, 0, :]
    inv = 1.0 / jnp.where(nv > 0.0, nv, 1.0)
    o_ref[...] = jnp.maximum(prod * inv[:, None] + b_ref[...], 0.0)


@jax.jit
def kernel(feats, inp_points, out_points, out_extents, scale_compat,
           neighbors_index, neighbors_row_splits, neighbors_distance, W, b):
    f32 = jnp.float32
    gtab = jnp.concatenate(
        [feats, inp_points,
         jnp.zeros((_N_IN, _GW - _C - 3), f32)], axis=1)
    rs = jnp.concatenate(
        [neighbors_row_splits,
         jnp.full((_RS_PAD - (_M + 1),), _E, jnp.int32)])
    eidx = jnp.concatenate(
        [neighbors_index, jnp.zeros((_E_PAD - _E,), jnp.int32)])
    esc = jnp.concatenate([scale_compat, jnp.zeros((_E_PAD - _E,), f32)])
    edist = jnp.concatenate(
        [neighbors_distance, jnp.zeros((_E_PAD - _E,), f32)])
    opts = jnp.zeros((_M_PAD, 16), f32).at[:_M, :3].set(out_points)
    ext = jnp.broadcast_to(out_extents, (16,))

    mesh = plsc.VectorSubcoreMesh(core_axis_name="c", subcore_axis_name="s")
    sc_call = pl.kernel(
        _sc_body,
        out_type=(jax.ShapeDtypeStruct((_M, _AW), f32),
                  jax.ShapeDtypeStruct((_M_PAD,), f32)),
        mesh=mesh,
        scratch_types=[
            pltpu.VMEM((_MPW + 16,), jnp.int32),
            pltpu.VMEM((_MPW, 16), f32),
            pltpu.VMEM((16,), f32),
            pltpu.VMEM((_EW,), jnp.int32),
            pltpu.VMEM((_EW,), f32),
            pltpu.VMEM((_EW,), f32),
            pltpu.VMEM((_CH,), jnp.int32),
            pltpu.VMEM((_CH, _GW), f32),
            pltpu.VMEM((_AW,), f32),
            pltpu.VMEM((8, _CH + 16), f32),
            pltpu.VMEM((8, _CH + 16), jnp.int32),
            pltpu.VMEM((_MPW + 16,), f32),
            pltpu.SemaphoreType.DMA,
        ],
        compiler_params=pltpu.CompilerParams(use_tc_tiling_on_sc=False, needs_layout_passes=False),
    )
    a_acc, norm = sc_call(gtab, rs, eidx, esc, edist, opts, ext)

    norm3 = norm[:_M].reshape(_M // _BM, 1, _BM)
    out = pl.pallas_call(
        _tc_body,
        grid=(_M // _BM,),
        in_specs=[
            pl.BlockSpec((_BM, _AW), lambda i: (i, 0)),
            pl.BlockSpec((_AW, _C), lambda i: (0, 0)),
            pl.BlockSpec((1, 1, _BM), lambda i: (i, 0, 0)),
            pl.BlockSpec((1, _C), lambda i: (0, 0)),
        ],
        out_specs=pl.BlockSpec((_BM, _C), lambda i: (i, 0)),
        out_shape=jax.ShapeDtypeStruct((_M, _C), f32),
    )(a_acc, W.reshape(_AW, _C), norm3, b.reshape(1, _C))
    return out


# speedup vs baseline: 3.7380x; 1.1781x over previous
import functools

import jax
import jax.numpy as jnp
from jax import lax
from jax.experimental import pallas as pl
from jax.experimental.pallas import tpu as pltpu
from jax.experimental.pallas import tpu_sc as plsc

_N_IN = 10000
_M = 5000
_E = 160000
_C = 128
_K = 4
_NCELL = _K * _K * _K
_AW = _NCELL * _C
_GW = 144
_NW = 32
_MPW = 160
_M_PAD = _NW * _MPW
_RS_PAD = (_NW - 1) * _MPW + _MPW + 8
_EW = 152
_E_PAD = _E + _EW
_CH = 128
_BM = 200


def _rsqrt(x):
    xi = plsc.bitcast(x, jnp.int32)
    y = plsc.bitcast(jnp.int32(0x5F3759DF) - (xi >> 1), jnp.float32)
    for _ in range(3):
        y = y * (1.5 - 0.5 * x * y * y)
    return y


def _sc_body(gtab, rs, eidx, esc, edist, opts, ext,
             a_out, norm_out,
             rs_v, op_v, ext_v, ew_idx, ew_sc, ew_dist, gidx, gbuf,
             acc, sstage, cstage, normbuf, sem):
    cid = lax.axis_index("c")
    sid = lax.axis_index("s")
    wid = sid * 2 + cid
    m0 = pl.multiple_of(wid * _MPW, _MPW)
    mcnt = jnp.minimum(_MPW, _M - m0)

    pltpu.sync_copy(rs.at[pl.ds(m0, _MPW + 8)], rs_v.at[pl.ds(0, _MPW + 8)])
    pltpu.sync_copy(opts.at[pl.ds(m0, _MPW)], op_v)
    pltpu.sync_copy(ext, ext_v)
    inv_h = 1.0 / (0.5 * ext_v[pl.ds(0, 16)])

    zero16 = jnp.zeros((16,), jnp.float32)
    iota16 = lax.iota(jnp.int32, 16)
    col_x = jnp.full((16,), _C, jnp.int32)
    col_y = jnp.full((16,), _C + 1, jnp.int32)
    col_z = jnp.full((16,), _C + 2, jnp.int32)

    def zero_acc():
        def zero_body(k, _):
            acc[pl.ds(k * 16, 16)] = zero16
            return 0
        lax.fori_loop(0, _AW // 16, zero_body, 0)

    def flush(dd, nacc):
        normbuf[pl.ds(dd, 16)] = jnp.full((16,), nacc)
        pltpu.sync_copy(acc, a_out.at[m0 + dd])
        zero_acc()

    e0 = rs_v[pl.ds(0, 16)][0]
    e1 = rs_v[pl.ds(mcnt, 16)][0]
    zero_acc()
    nchunks = (e1 - e0 + (_CH - 1)) >> 7

    def chunk_body(ch, carry):
        dd, nacc = carry
        c0 = e0 + ch * _CH
        cl = jnp.minimum(e1 - c0, _CH)
        a0 = pl.multiple_of(c0 & ~jnp.int32(7), 8)
        off = c0 - a0
        pltpu.sync_copy(eidx.at[pl.ds(a0, _EW)], ew_idx)
        pltpu.sync_copy(esc.at[pl.ds(a0, _EW)], ew_sc)
        pltpu.sync_copy(edist.at[pl.ds(a0, _EW)], ew_dist)

        def cp_body(g, _):
            gidx[pl.ds(g * 16, 16)] = ew_idx[pl.ds(off + g * 16, 16)]
            return 0
        lax.fori_loop(0, _CH // 16, cp_body, 0)

        pltpu.async_copy(gtab.at[gidx], gbuf, sem).wait()
        cend = c0 + cl

        def process_seg(la, lb, dd2):
            opv = op_v[dd2, pl.ds(0, 16)]
            ox, oy, oz = opv[0], opv[1], opv[2]
            ng = (lb - la + 15) >> 4

            def seg_group(g, na):
                lane0 = la + g * 16
                lanes = iota16 + lane0
                msk = lanes < lb
                lg = jnp.minimum(lanes, _CH - 1)
                x = plsc.load_gather(gbuf, [lg, col_x])
                y = plsc.load_gather(gbuf, [lg, col_y])
                z = plsc.load_gather(gbuf, [lg, col_z])
                rx = (x - ox) * inv_h
                ry = (y - oy) * inv_h
                rz = (z - oz) * inv_h
                r2sq = rx * rx + ry * ry + rz * rz + 1e-12
                ih = _rsqrt(r2sq)
                r2 = r2sq * ih
                rinf = jnp.maximum(jnp.maximum(jnp.abs(rx), jnp.abs(ry)),
                                   jnp.abs(rz))
                scale = jnp.where(rinf > 1e-12,
                                  r2 / jnp.maximum(rinf, 1e-12), 0.0)
                cx = jnp.clip(rx * scale, -1.0, 1.0)
                cy = jnp.clip(ry * scale, -1.0, 1.0)
                cz = jnp.clip(rz * scale, -1.0, 1.0)
                gx = (cx * 0.5 + 0.5) * (_K - 1.0)
                gy = (cy * 0.5 + 0.5) * (_K - 1.0)
                gz = (cz * 0.5 + 0.5) * (_K - 1.0)
                g0x = jnp.minimum(gx.astype(jnp.int32), _K - 2)
                g0y = jnp.minimum(gy.astype(jnp.int32), _K - 2)
                g0z = jnp.minimum(gz.astype(jnp.int32), _K - 2)
                fx = gx - g0x.astype(jnp.float32)
                fy = gy - g0y.astype(jnp.float32)
                fz = gz - g0z.astype(jnp.float32)
                scv = ew_sc[pl.ds(off + lane0, 16)]
                dv = ew_dist[pl.ds(off + lane0, 16)]
                rsq = jnp.clip(dv, 0.0, 1.0)
                t1 = 1.0 - rsq
                imp = jnp.where(msk, scv * (t1 * t1 * t1), 0.0)
                na = na + jnp.sum(imp)
                wx = (1.0 - fx, fx)
                wy = (1.0 - fy, fy)
                wz = (1.0 - fz, fz)
                for c in range(8):
                    bx, by, bz = c & 1, (c >> 1) & 1, (c >> 2) & 1
                    sv = imp * (wx[bx] * wy[by] * wz[bz])
                    cell = ((g0x + bx) * (_K * _K) + (g0y + by) * _K
                            + (g0z + bz))
                    sstage[c, pl.ds(lane0, 16)] = sv
                    cstage[c, pl.ds(lane0, 16)] = cell
                lend = jnp.minimum(lane0 + 16, lb)

                def edge_i(i, _):
                    fvs = [gbuf[i, pl.ds(j * 16, 16)] for j in range(8)]
                    for c in range(8):
                        sv = jnp.full((16,), sstage[c, pl.ds(i, 16)][0])
                        base = cstage[c, pl.ds(i, 16)][0] * _C
                        for j in range(8):
                            plsc.addupdate(
                                acc.at[pl.ds(base + j * 16, 16)],
                                fvs[j] * sv)
                    return 0
                lax.fori_loop(lane0, lend, edge_i, 0)
                return na
            return lax.fori_loop(0, ng, seg_group, jnp.float32(0.0))

        def wcond(c2):
            dd2, _ = c2
            return jnp.logical_and(dd2 < mcnt,
                                   rs_v[pl.ds(dd2 + 1, 16)][0] <= cend)

        def wbody(c2):
            dd2, na2 = c2
            s = rs_v[pl.ds(dd2, 16)][0]
            t = rs_v[pl.ds(dd2 + 1, 16)][0]
            la = jnp.maximum(s, c0) - c0
            na2 = na2 + process_seg(la, t - c0, dd2)
            flush(dd2, na2)
            return (dd2 + 1, jnp.float32(0.0))

        dd, nacc = lax.while_loop(wcond, wbody, (dd, nacc))

        ddc = jnp.minimum(dd, _MPW - 1)
        s = rs_v[pl.ds(ddc, 16)][0]

        def partial(na2):
            la = jnp.maximum(s, c0) - c0
            return na2 + process_seg(la, cl, ddc)

        nacc = lax.cond(jnp.logical_and(dd < mcnt, s < cend),
                        partial, lambda na2: na2, nacc)
        return (dd, nacc)

    dd, nacc = lax.fori_loop(0, nchunks, chunk_body,
                             (jnp.int32(0), jnp.float32(0.0)))

    def tail_cond(c2):
        dd2, _ = c2
        return dd2 < mcnt

    def tail_body(c2):
        dd2, na2 = c2
        flush(dd2, na2)
        return (dd2 + 1, jnp.float32(0.0))

    lax.while_loop(tail_cond, tail_body, (dd, nacc))
    pltpu.sync_copy(normbuf.at[pl.ds(0, _MPW)], norm_out.at[pl.ds(m0, _MPW)])


def _tc_body(a_ref, w_ref, n_ref, b_ref, o_ref):
    prod = jnp.dot(a_ref[...], w_ref[...],
                   preferred_element_type=jnp.float32)
    nv = n_ref[0, 0, :]
    inv = 1.0 / jnp.where(nv > 0.0, nv, 1.0)
    o_ref[...] = jnp.maximum(prod * inv[:, None] + b_ref[...], 0.0)


@jax.jit
def kernel(feats, inp_points, out_points, out_extents, scale_compat,
           neighbors_index, neighbors_row_splits, neighbors_distance, W, b):
    f32 = jnp.float32
    gtab = jnp.concatenate(
        [feats, inp_points,
         jnp.zeros((_N_IN, _GW - _C - 3), f32)], axis=1)
    rs = jnp.concatenate(
        [neighbors_row_splits,
         jnp.full((_RS_PAD - (_M + 1),), _E, jnp.int32)])
    eidx = jnp.concatenate(
        [neighbors_index, jnp.zeros((_E_PAD - _E,), jnp.int32)])
    esc = jnp.concatenate([scale_compat, jnp.zeros((_E_PAD - _E,), f32)])
    edist = jnp.concatenate(
        [neighbors_distance, jnp.zeros((_E_PAD - _E,), f32)])
    opts = jnp.zeros((_M_PAD, 16), f32).at[:_M, :3].set(out_points)
    ext = jnp.broadcast_to(out_extents, (16,))

    mesh = plsc.VectorSubcoreMesh(core_axis_name="c", subcore_axis_name="s")
    sc_call = pl.kernel(
        _sc_body,
        out_type=(jax.ShapeDtypeStruct((_M, _AW), f32),
                  jax.ShapeDtypeStruct((_M_PAD,), f32)),
        mesh=mesh,
        scratch_types=[
            pltpu.VMEM((_MPW + 16,), jnp.int32),
            pltpu.VMEM((_MPW, 16), f32),
            pltpu.VMEM((16,), f32),
            pltpu.VMEM((_EW,), jnp.int32),
            pltpu.VMEM((_EW,), f32),
            pltpu.VMEM((_EW,), f32),
            pltpu.VMEM((_CH,), jnp.int32),
            pltpu.VMEM((_CH, _GW), f32),
            pltpu.VMEM((_AW,), f32),
            pltpu.VMEM((8, _CH + 16), f32),
            pltpu.VMEM((8, _CH + 16), jnp.int32),
            pltpu.VMEM((_MPW + 16,), f32),
            pltpu.SemaphoreType.DMA,
        ],
        compiler_params=pltpu.CompilerParams(use_tc_tiling_on_sc=False, needs_layout_passes=False),
    )
    a_acc, norm = sc_call(gtab, rs, eidx, esc, edist, opts, ext)

    norm3 = norm[:_M].reshape(_M // _BM, 1, _BM)
    out = pl.pallas_call(
        _tc_body,
        grid=(_M // _BM,),
        in_specs=[
            pl.BlockSpec((_BM, _AW), lambda i: (i, 0)),
            pl.BlockSpec((_AW, _C), lambda i: (0, 0)),
            pl.BlockSpec((1, 1, _BM), lambda i: (i, 0, 0)),
            pl.BlockSpec((1, _C), lambda i: (0, 0)),
        ],
        out_specs=pl.BlockSpec((_BM, _C), lambda i: (i, 0)),
        out_shape=jax.ShapeDtypeStruct((_M, _C), f32),
    )(a_acc, W.reshape(_AW, _C), norm3, b.reshape(1, _C))
    return out


# speedup vs baseline: 5.5705x; 1.4902x over previous
import functools

import jax
import jax.numpy as jnp
from jax import lax
from jax.experimental import pallas as pl
from jax.experimental.pallas import tpu as pltpu
from jax.experimental.pallas import tpu_sc as plsc

_N_IN = 10000
_M = 5000
_E = 160000
_C = 128
_K = 4
_NCELL = _K * _K * _K
_AW = _NCELL * _C
_GW = 144
_NW = 32
_MPW = 160
_M_PAD = _NW * _MPW
_RS_PAD = (_NW - 1) * _MPW + _MPW + 8
_EW = 152
_E_PAD = _E + _EW
_CH = 128
_BM = 200


_GDN = lax.GatherDimensionNumbers(offset_dims=(), collapsed_slice_dims=(0,),
                                  start_index_map=(0,))


def _take16(v, idxv):
    return lax.gather(v, idxv[:, None], _GDN, (1,),
                      mode=lax.GatherScatterMode.PROMISE_IN_BOUNDS)


def _rsqrt(x):
    xi = plsc.bitcast(x, jnp.int32)
    y = plsc.bitcast(jnp.int32(0x5F3759DF) - (xi >> 1), jnp.float32)
    for _ in range(3):
        y = y * (1.5 - 0.5 * x * y * y)
    return y


def _sc_body(gtab, rs, eidx, esc, edist, opts, ext,
             a_out, norm_out,
             rs_v, op_v, ext_v, ew_idx, ew_sc, ew_dist, gidx, gbuf,
             acc, sstage, cstage, normbuf, sem):
    cid = lax.axis_index("c")
    sid = lax.axis_index("s")
    wid = sid * 2 + cid
    m0 = pl.multiple_of(wid * _MPW, _MPW)
    mcnt = jnp.minimum(_MPW, _M - m0)

    pltpu.sync_copy(rs.at[pl.ds(m0, _MPW + 8)], rs_v.at[pl.ds(0, _MPW + 8)])
    pltpu.sync_copy(opts.at[pl.ds(m0, _MPW)], op_v)
    pltpu.sync_copy(ext, ext_v)
    inv_h = 1.0 / (0.5 * ext_v[pl.ds(0, 16)])

    zero16 = jnp.zeros((16,), jnp.float32)
    iota16 = lax.iota(jnp.int32, 16)
    col_x = jnp.full((16,), _C, jnp.int32)
    col_y = jnp.full((16,), _C + 1, jnp.int32)
    col_z = jnp.full((16,), _C + 2, jnp.int32)

    def zero_acc():
        def zero_body(k, _):
            acc[pl.ds(k * 16, 16)] = zero16
            return 0
        lax.fori_loop(0, _AW // 16, zero_body, 0)

    def flush(dd, nacc):
        normbuf[pl.ds(dd, 16)] = jnp.full((16,), nacc)
        pltpu.sync_copy(acc, a_out.at[m0 + dd])
        zero_acc()

    e0 = rs_v[pl.ds(0, 16)][0]
    e1 = rs_v[pl.ds(mcnt, 16)][0]
    zero_acc()
    nchunks = (e1 - e0 + (_CH - 1)) >> 7

    def chunk_body(ch, carry):
        dd, nacc = carry
        c0 = e0 + ch * _CH
        cl = jnp.minimum(e1 - c0, _CH)
        a0 = pl.multiple_of(c0 & ~jnp.int32(7), 8)
        off = c0 - a0
        pltpu.sync_copy(eidx.at[pl.ds(a0, _EW)], ew_idx)
        pltpu.sync_copy(esc.at[pl.ds(a0, _EW)], ew_sc)
        pltpu.sync_copy(edist.at[pl.ds(a0, _EW)], ew_dist)

        def cp_body(g, _):
            gidx[pl.ds(g * 16, 16)] = ew_idx[pl.ds(off + g * 16, 16)]
            return 0
        lax.fori_loop(0, _CH // 16, cp_body, 0)

        pltpu.async_copy(gtab.at[gidx], gbuf, sem).wait()
        cend = c0 + cl

        def process_seg(la, lb, dd2):
            opv = op_v[dd2, pl.ds(0, 16)]
            ox, oy, oz = opv[0], opv[1], opv[2]
            ng = (lb - la + 15) >> 4

            def seg_group(g, na):
                lane0 = la + g * 16
                lanes = iota16 + lane0
                msk = lanes < lb
                lg = jnp.minimum(lanes, _CH - 1)
                x = plsc.load_gather(gbuf, [lg, col_x])
                y = plsc.load_gather(gbuf, [lg, col_y])
                z = plsc.load_gather(gbuf, [lg, col_z])
                rx = (x - ox) * inv_h
                ry = (y - oy) * inv_h
                rz = (z - oz) * inv_h
                r2sq = rx * rx + ry * ry + rz * rz + 1e-12
                ih = _rsqrt(r2sq)
                r2 = r2sq * ih
                rinf = jnp.maximum(jnp.maximum(jnp.abs(rx), jnp.abs(ry)),
                                   jnp.abs(rz))
                scale = jnp.where(rinf > 1e-12,
                                  r2 / jnp.maximum(rinf, 1e-12), 0.0)
                cx = jnp.clip(rx * scale, -1.0, 1.0)
                cy = jnp.clip(ry * scale, -1.0, 1.0)
                cz = jnp.clip(rz * scale, -1.0, 1.0)
                gx = (cx * 0.5 + 0.5) * (_K - 1.0)
                gy = (cy * 0.5 + 0.5) * (_K - 1.0)
                gz = (cz * 0.5 + 0.5) * (_K - 1.0)
                g0x = jnp.minimum(gx.astype(jnp.int32), _K - 2)
                g0y = jnp.minimum(gy.astype(jnp.int32), _K - 2)
                g0z = jnp.minimum(gz.astype(jnp.int32), _K - 2)
                fx = gx - g0x.astype(jnp.float32)
                fy = gy - g0y.astype(jnp.float32)
                fz = gz - g0z.astype(jnp.float32)
                scv = ew_sc[pl.ds(off + lane0, 16)]
                dv = ew_dist[pl.ds(off + lane0, 16)]
                rsq = jnp.clip(dv, 0.0, 1.0)
                t1 = 1.0 - rsq
                imp = jnp.where(msk, scv * (t1 * t1 * t1), 0.0)
                na = na + jnp.sum(imp)
                wx = (1.0 - fx, fx)
                wy = (1.0 - fy, fy)
                wz = (1.0 - fz, fz)
                svs = []
                cbs = []
                for c in range(8):
                    bx, by, bz = c & 1, (c >> 1) & 1, (c >> 2) & 1
                    svs.append(imp * (wx[bx] * wy[by] * wz[bz]))
                    cbs.append(((g0x + bx) * (_K * _K) + (g0y + by) * _K
                                + (g0z + bz)) * _C)
                lend = jnp.minimum(lane0 + 16, lb)

                def edge_i(i, _):
                    ilv = jnp.full((16,), i - lane0, jnp.int32)
                    fvs = [gbuf[i, pl.ds(j * 16, 16)] for j in range(8)]
                    for c in range(8):
                        svb = _take16(svs[c], ilv)
                        base = _take16(cbs[c], ilv)[0]
                        for j in range(8):
                            plsc.addupdate(
                                acc.at[pl.ds(base + j * 16, 16)],
                                fvs[j] * svb)
                    return 0
                lax.fori_loop(lane0, lend, edge_i, 0)
                return na
            return lax.fori_loop(0, ng, seg_group, jnp.float32(0.0))

        def wcond(c2):
            dd2, _ = c2
            return jnp.logical_and(dd2 < mcnt,
                                   rs_v[pl.ds(dd2 + 1, 16)][0] <= cend)

        def wbody(c2):
            dd2, na2 = c2
            s = rs_v[pl.ds(dd2, 16)][0]
            t = rs_v[pl.ds(dd2 + 1, 16)][0]
            la = jnp.maximum(s, c0) - c0
            na2 = na2 + process_seg(la, t - c0, dd2)
            flush(dd2, na2)
            return (dd2 + 1, jnp.float32(0.0))

        dd, nacc = lax.while_loop(wcond, wbody, (dd, nacc))

        ddc = jnp.minimum(dd, _MPW - 1)
        s = rs_v[pl.ds(ddc, 16)][0]

        def partial(na2):
            la = jnp.maximum(s, c0) - c0
            return na2 + process_seg(la, cl, ddc)

        nacc = lax.cond(jnp.logical_and(dd < mcnt, s < cend),
                        partial, lambda na2: na2, nacc)
        return (dd, nacc)

    dd, nacc = lax.fori_loop(0, nchunks, chunk_body,
                             (jnp.int32(0), jnp.float32(0.0)))

    def tail_cond(c2):
        dd2, _ = c2
        return dd2 < mcnt

    def tail_body(c2):
        dd2, na2 = c2
        flush(dd2, na2)
        return (dd2 + 1, jnp.float32(0.0))

    lax.while_loop(tail_cond, tail_body, (dd, nacc))
    pltpu.sync_copy(normbuf.at[pl.ds(0, _MPW)], norm_out.at[pl.ds(m0, _MPW)])


def _tc_body(a_ref, w_ref, n_ref, b_ref, o_ref):
    prod = jnp.dot(a_ref[...], w_ref[...],
                   preferred_element_type=jnp.float32)
    nv = n_ref[0, 0, :]
    inv = 1.0 / jnp.where(nv > 0.0, nv, 1.0)
    o_ref[...] = jnp.maximum(prod * inv[:, None] + b_ref[...], 0.0)


@jax.jit
def kernel(feats, inp_points, out_points, out_extents, scale_compat,
           neighbors_index, neighbors_row_splits, neighbors_distance, W, b):
    f32 = jnp.float32
    gtab = jnp.concatenate(
        [feats, inp_points,
         jnp.zeros((_N_IN, _GW - _C - 3), f32)], axis=1)
    rs = jnp.concatenate(
        [neighbors_row_splits,
         jnp.full((_RS_PAD - (_M + 1),), _E, jnp.int32)])
    eidx = jnp.concatenate(
        [neighbors_index, jnp.zeros((_E_PAD - _E,), jnp.int32)])
    esc = jnp.concatenate([scale_compat, jnp.zeros((_E_PAD - _E,), f32)])
    edist = jnp.concatenate(
        [neighbors_distance, jnp.zeros((_E_PAD - _E,), f32)])
    opts = jnp.zeros((_M_PAD, 16), f32).at[:_M, :3].set(out_points)
    ext = jnp.broadcast_to(out_extents, (16,))

    mesh = plsc.VectorSubcoreMesh(core_axis_name="c", subcore_axis_name="s")
    sc_call = pl.kernel(
        _sc_body,
        out_type=(jax.ShapeDtypeStruct((_M, _AW), f32),
                  jax.ShapeDtypeStruct((_M_PAD,), f32)),
        mesh=mesh,
        scratch_types=[
            pltpu.VMEM((_MPW + 16,), jnp.int32),
            pltpu.VMEM((_MPW, 16), f32),
            pltpu.VMEM((16,), f32),
            pltpu.VMEM((_EW,), jnp.int32),
            pltpu.VMEM((_EW,), f32),
            pltpu.VMEM((_EW,), f32),
            pltpu.VMEM((_CH,), jnp.int32),
            pltpu.VMEM((_CH, _GW), f32),
            pltpu.VMEM((_AW,), f32),
            pltpu.VMEM((8, _CH + 16), f32),
            pltpu.VMEM((8, _CH + 16), jnp.int32),
            pltpu.VMEM((_MPW + 16,), f32),
            pltpu.SemaphoreType.DMA,
        ],
        compiler_params=pltpu.CompilerParams(use_tc_tiling_on_sc=False, needs_layout_passes=False),
    )
    a_acc, norm = sc_call(gtab, rs, eidx, esc, edist, opts, ext)

    norm3 = norm[:_M].reshape(_M // _BM, 1, _BM)
    out = pl.pallas_call(
        _tc_body,
        grid=(_M // _BM,),
        in_specs=[
            pl.BlockSpec((_BM, _AW), lambda i: (i, 0)),
            pl.BlockSpec((_AW, _C), lambda i: (0, 0)),
            pl.BlockSpec((1, 1, _BM), lambda i: (i, 0, 0)),
            pl.BlockSpec((1, _C), lambda i: (0, 0)),
        ],
        out_specs=pl.BlockSpec((_BM, _C), lambda i: (i, 0)),
        out_shape=jax.ShapeDtypeStruct((_M, _C), f32),
    )(a_acc, W.reshape(_AW, _C), norm3, b.reshape(1, _C))
    return out


# speedup vs baseline: 6.1633x; 1.1064x over previous
import functools

import jax
import jax.numpy as jnp
from jax import lax
from jax.experimental import pallas as pl
from jax.experimental.pallas import tpu as pltpu
from jax.experimental.pallas import tpu_sc as plsc

_N_IN = 10000
_M = 5000
_E = 160000
_C = 128
_K = 4
_NCELL = _K * _K * _K
_AW = _NCELL * _C
_GW = 144
_NW = 32
_MPW = 160
_M_PAD = _NW * _MPW
_RS_PAD = (_NW - 1) * _MPW + _MPW + 8
_EW = 152
_E_PAD = _E + _EW
_CH = 128
_BM = 200


_GDN = lax.GatherDimensionNumbers(offset_dims=(), collapsed_slice_dims=(0,),
                                  start_index_map=(0,))


def _take16(v, idxv):
    return lax.gather(v, idxv[:, None], _GDN, (1,),
                      mode=lax.GatherScatterMode.PROMISE_IN_BOUNDS)


def _rsqrt(x):
    xi = plsc.bitcast(x, jnp.int32)
    y = plsc.bitcast(jnp.int32(0x5F3759DF) - (xi >> 1), jnp.float32)
    for _ in range(3):
        y = y * (1.5 - 0.5 * x * y * y)
    return y


def _sc_body(gtab, rs, eidx, esc, edist, opts, ext,
             a_out, norm_out,
             rs_v, op_v, ext_v,
             ew_idx0, ew_sc0, ew_dist0, gidx0, gbuf0,
             ew_idx1, ew_sc1, ew_dist1, gidx1, gbuf1,
             acc, normbuf, semg0, semg1, semf):
    cid = lax.axis_index("c")
    sid = lax.axis_index("s")
    wid = sid * 2 + cid
    m0 = pl.multiple_of(wid * _MPW, _MPW)
    mcnt = jnp.minimum(_MPW, _M - m0)

    pltpu.sync_copy(rs.at[pl.ds(m0, _MPW + 8)], rs_v.at[pl.ds(0, _MPW + 8)])
    pltpu.sync_copy(opts.at[pl.ds(m0, _MPW)], op_v)
    pltpu.sync_copy(ext, ext_v)
    inv_h = 1.0 / (0.5 * ext_v[pl.ds(0, 16)])

    zero16 = jnp.zeros((16,), jnp.float32)
    iota16 = lax.iota(jnp.int32, 16)
    col_x = jnp.full((16,), _C, jnp.int32)
    col_y = jnp.full((16,), _C + 1, jnp.int32)
    col_z = jnp.full((16,), _C + 2, jnp.int32)

    def zero_acc(qoff):
        def zero_body(k, _):
            acc[pl.ds(qoff + k * 16, 16)] = zero16
            return 0
        lax.fori_loop(0, _AW // 16, zero_body, 0)

    def flush(dd, nacc):
        normbuf[pl.ds(dd, 16)] = jnp.full((16,), nacc)
        qoff = pl.multiple_of((dd & 1) * _AW, 8)

        def drain(_):
            pltpu.make_async_copy(acc.at[pl.ds(0, _AW)], a_out.at[m0],
                                  semf).wait()
            return 0
        lax.cond(dd >= 1, drain, lambda _: 0, 0)
        pltpu.async_copy(acc.at[pl.ds(qoff, _AW)], a_out.at[m0 + dd], semf)
        zero_acc(pl.multiple_of(_AW - qoff, 8))

    e0 = rs_v[pl.ds(0, 16)][0]
    e1 = rs_v[pl.ds(mcnt, 16)][0]
    zero_acc(0)
    zero_acc(_AW)
    nchunks = (e1 - e0 + (_CH - 1)) >> 7

    def prefetch(ch, ewi, ews, ewd, gix, gbf, smg):
        def go(_):
            c0 = e0 + ch * _CH
            a0 = pl.multiple_of(c0 & ~jnp.int32(7), 8)
            off = c0 - a0
            pltpu.sync_copy(eidx.at[pl.ds(a0, _EW)], ewi)
            pltpu.sync_copy(esc.at[pl.ds(a0, _EW)], ews)
            pltpu.sync_copy(edist.at[pl.ds(a0, _EW)], ewd)

            def cp_body(g, _):
                gix[pl.ds(g * 16, 16)] = ewi[pl.ds(off + g * 16, 16)]
                return 0
            lax.fori_loop(0, _CH // 16, cp_body, 0)
            pltpu.async_copy(gtab.at[gix], gbf, smg)
            return 0
        lax.cond(ch < nchunks, go, lambda _: 0, 0)

    def chunk(ch, carry, ews, ewd, gix, gbf, smg):
        dd0, nacc0 = carry
        c0 = e0 + ch * _CH
        cl = jnp.minimum(e1 - c0, _CH)
        a0 = pl.multiple_of(c0 & ~jnp.int32(7), 8)
        off = c0 - a0
        cend = c0 + cl

        def process_seg(la, lb, dd2):
            opv = op_v[dd2, pl.ds(0, 16)]
            ox, oy, oz = opv[0], opv[1], opv[2]
            qoff = (dd2 & 1) * _AW
            ng = (lb - la + 15) >> 4

            def seg_group(g, na):
                lane0 = la + g * 16
                lanes = iota16 + lane0
                msk = lanes < lb
                lg = jnp.minimum(lanes, _CH - 1)
                x = plsc.load_gather(gbf, [lg, col_x])
                y = plsc.load_gather(gbf, [lg, col_y])
                z = plsc.load_gather(gbf, [lg, col_z])
                rx = (x - ox) * inv_h
                ry = (y - oy) * inv_h
                rz = (z - oz) * inv_h
                r2sq = rx * rx + ry * ry + rz * rz + 1e-12
                ih = _rsqrt(r2sq)
                r2 = r2sq * ih
                rinf = jnp.maximum(jnp.maximum(jnp.abs(rx), jnp.abs(ry)),
                                   jnp.abs(rz))
                scale = jnp.where(rinf > 1e-12,
                                  r2 / jnp.maximum(rinf, 1e-12), 0.0)
                cx = jnp.clip(rx * scale, -1.0, 1.0)
                cy = jnp.clip(ry * scale, -1.0, 1.0)
                cz = jnp.clip(rz * scale, -1.0, 1.0)
                gx = (cx * 0.5 + 0.5) * (_K - 1.0)
                gy = (cy * 0.5 + 0.5) * (_K - 1.0)
                gz = (cz * 0.5 + 0.5) * (_K - 1.0)
                g0x = jnp.minimum(gx.astype(jnp.int32), _K - 2)
                g0y = jnp.minimum(gy.astype(jnp.int32), _K - 2)
                g0z = jnp.minimum(gz.astype(jnp.int32), _K - 2)
                fx = gx - g0x.astype(jnp.float32)
                fy = gy - g0y.astype(jnp.float32)
                fz = gz - g0z.astype(jnp.float32)
                scv = ews[pl.ds(off + lane0, 16)]
                dv = ewd[pl.ds(off + lane0, 16)]
                rsq = jnp.clip(dv, 0.0, 1.0)
                t1 = 1.0 - rsq
                imp = jnp.where(msk, scv * (t1 * t1 * t1), 0.0)
                na = na + jnp.sum(imp)
                wx = (1.0 - fx, fx)
                wy = (1.0 - fy, fy)
                wz = (1.0 - fz, fz)
                svs = []
                cbs = []
                for c in range(8):
                    bx, by, bz = c & 1, (c >> 1) & 1, (c >> 2) & 1
                    svs.append(imp * (wx[bx] * wy[by] * wz[bz]))
                    cbs.append(((g0x + bx) * (_K * _K) + (g0y + by) * _K
                                + (g0z + bz)) * _C)
                lend = jnp.minimum(lane0 + 16, lb)

                def edge_i(i, _):
                    ilv = jnp.full((16,), i - lane0, jnp.int32)
                    fvs = [gbf[i, pl.ds(j * 16, 16)] for j in range(8)]
                    for c in range(8):
                        svb = _take16(svs[c], ilv)
                        base = qoff + _take16(cbs[c], ilv)[0]
                        for j in range(8):
                            plsc.addupdate(
                                acc.at[pl.ds(base + j * 16, 16)],
                                fvs[j] * svb)
                    return 0
                lax.fori_loop(lane0, lend, edge_i, 0)
                return na
            return lax.fori_loop(0, ng, seg_group, jnp.float32(0.0))

        def wcond(c2):
            dd2, _ = c2
            return jnp.logical_and(dd2 < mcnt,
                                   rs_v[pl.ds(dd2 + 1, 16)][0] <= cend)

        def wbody(c2):
            dd2, na2 = c2
            s = rs_v[pl.ds(dd2, 16)][0]
            t = rs_v[pl.ds(dd2 + 1, 16)][0]
            la = jnp.maximum(s, c0) - c0
            na2 = na2 + process_seg(la, t - c0, dd2)
            flush(dd2, na2)
            return (dd2 + 1, jnp.float32(0.0))

        def go(carry2):
            dd, nacc = carry2
            pltpu.make_async_copy(gtab.at[gix], gbf, smg).wait()
            dd, nacc = lax.while_loop(wcond, wbody, (dd, nacc))
            ddc = jnp.minimum(dd, _MPW - 1)
            s = rs_v[pl.ds(ddc, 16)][0]

            def partial(na2):
                la = jnp.maximum(s, c0) - c0
                return na2 + process_seg(la, cl, ddc)

            nacc = lax.cond(jnp.logical_and(dd < mcnt, s < cend),
                            partial, lambda na2: na2, nacc)
            return (dd, nacc)

        return lax.cond(ch < nchunks, go, lambda c2: c2, (dd0, nacc0))

    prefetch(jnp.int32(0), ew_idx0, ew_sc0, ew_dist0, gidx0, gbuf0, semg0)
    npairs = (nchunks + 1) >> 1

    def pair_body(pp, carry):
        ch0 = pp * 2
        prefetch(ch0 + 1, ew_idx1, ew_sc1, ew_dist1, gidx1, gbuf1, semg1)
        carry = chunk(ch0, carry, ew_sc0, ew_dist0, gidx0, gbuf0, semg0)
        prefetch(ch0 + 2, ew_idx0, ew_sc0, ew_dist0, gidx0, gbuf0, semg0)
        carry = chunk(ch0 + 1, carry, ew_sc1, ew_dist1, gidx1, gbuf1, semg1)
        return carry

    dd, nacc = lax.fori_loop(0, npairs, pair_body,
                             (jnp.int32(0), jnp.float32(0.0)))

    def tail_cond(c2):
        dd2, _ = c2
        return dd2 < mcnt

    def tail_body(c2):
        dd2, na2 = c2
        flush(dd2, na2)
        return (dd2 + 1, jnp.float32(0.0))

    lax.while_loop(tail_cond, tail_body, (dd, nacc))
    pltpu.make_async_copy(acc.at[pl.ds(0, _AW)], a_out.at[m0], semf).wait()
    pltpu.sync_copy(normbuf.at[pl.ds(0, _MPW)], norm_out.at[pl.ds(m0, _MPW)])


def _tc_body(a_ref, w_ref, n_ref, b_ref, o_ref):
    prod = jnp.dot(a_ref[...], w_ref[...],
                   preferred_element_type=jnp.float32)
    nv = n_ref[0, 0, :]
    inv = 1.0 / jnp.where(nv > 0.0, nv, 1.0)
    o_ref[...] = jnp.maximum(prod * inv[:, None] + b_ref[...], 0.0)


@jax.jit
def kernel(feats, inp_points, out_points, out_extents, scale_compat,
           neighbors_index, neighbors_row_splits, neighbors_distance, W, b):
    f32 = jnp.float32
    gtab = jnp.concatenate(
        [feats, inp_points,
         jnp.zeros((_N_IN, _GW - _C - 3), f32)], axis=1)
    rs = jnp.concatenate(
        [neighbors_row_splits,
         jnp.full((_RS_PAD - (_M + 1),), _E, jnp.int32)])
    eidx = jnp.concatenate(
        [neighbors_index, jnp.zeros((_E_PAD - _E,), jnp.int32)])
    esc = jnp.concatenate([scale_compat, jnp.zeros((_E_PAD - _E,), f32)])
    edist = jnp.concatenate(
        [neighbors_distance, jnp.zeros((_E_PAD - _E,), f32)])
    opts = jnp.zeros((_M_PAD, 16), f32).at[:_M, :3].set(out_points)
    ext = jnp.broadcast_to(out_extents, (16,))

    mesh = plsc.VectorSubcoreMesh(core_axis_name="c", subcore_axis_name="s")
    sc_call = pl.kernel(
        _sc_body,
        out_type=(jax.ShapeDtypeStruct((_M, _AW), f32),
                  jax.ShapeDtypeStruct((_M_PAD,), f32)),
        mesh=mesh,
        scratch_types=[
            pltpu.VMEM((_MPW + 16,), jnp.int32),
            pltpu.VMEM((_MPW, 16), f32),
            pltpu.VMEM((16,), f32),
            pltpu.VMEM((_EW,), jnp.int32),
            pltpu.VMEM((_EW,), f32),
            pltpu.VMEM((_EW,), f32),
            pltpu.VMEM((_CH,), jnp.int32),
            pltpu.VMEM((_CH, _GW), f32),
            pltpu.VMEM((_EW,), jnp.int32),
            pltpu.VMEM((_EW,), f32),
            pltpu.VMEM((_EW,), f32),
            pltpu.VMEM((_CH,), jnp.int32),
            pltpu.VMEM((_CH, _GW), f32),
            pltpu.VMEM((2 * _AW,), f32),
            pltpu.VMEM((_MPW + 16,), f32),
            pltpu.SemaphoreType.DMA,
            pltpu.SemaphoreType.DMA,
            pltpu.SemaphoreType.DMA,
        ],
        compiler_params=pltpu.CompilerParams(use_tc_tiling_on_sc=False, needs_layout_passes=False),
    )
    a_acc, norm = sc_call(gtab, rs, eidx, esc, edist, opts, ext)

    norm3 = norm[:_M].reshape(_M // _BM, 1, _BM)
    out = pl.pallas_call(
        _tc_body,
        grid=(_M // _BM,),
        in_specs=[
            pl.BlockSpec((_BM, _AW), lambda i: (i, 0)),
            pl.BlockSpec((_AW, _C), lambda i: (0, 0)),
            pl.BlockSpec((1, 1, _BM), lambda i: (i, 0, 0)),
            pl.BlockSpec((1, _C), lambda i: (0, 0)),
        ],
        out_specs=pl.BlockSpec((_BM, _C), lambda i: (i, 0)),
        out_shape=jax.ShapeDtypeStruct((_M, _C), f32),
    )(a_acc, W.reshape(_AW, _C), norm3, b.reshape(1, _C))
    return out
